# baseline (speedup 1.0000x reference)
import functools

import jax
import jax.numpy as jnp
import numpy as np
from jax import lax
from jax.experimental import pallas as pl
from jax.experimental.pallas import tpu as pltpu
from jax.experimental.pallas import tpu_sc as plsc

N_MOL = 256
BS = 128
D = 128
NEG_SLOPE = 0.01
NC = 2
NS = 16
NW = NC * NS


def _mesh():
    return plsc.VectorSubcoreMesh(core_axis_name="c", subcore_axis_name="s")


def _make_inner_kernel(N, E, C):
    EPW = E // NW
    NPW = N // NS
    assert NPW % 8 == 0 and EPW % C == 0
    n_chunks = EPW // C

    @functools.partial(
        pl.kernel,
        mesh=_mesh(),
        compiler_params=pltpu.CompilerParams(needs_layout_passes=False),
        out_type=jax.ShapeDtypeStruct((NC, N, D), jnp.float32),
        scratch_types=[
            pltpu.VMEM((C,), jnp.int32), pltpu.VMEM((C,), jnp.int32),
            pltpu.VMEM((C,), jnp.int32), pltpu.VMEM((C,), jnp.int32),
            pltpu.VMEM((C,), jnp.int32), pltpu.VMEM((C,), jnp.int32),
            pltpu.VMEM((C, D), jnp.float32), pltpu.VMEM((C, D), jnp.float32),
            pltpu.VMEM((C, D), jnp.float32), pltpu.VMEM((C, D), jnp.float32),
            pltpu.VMEM_SHARED((N, D), jnp.float32),
            pltpu.SemaphoreType.DMA, pltpu.SemaphoreType.DMA,
        ],
    )
    def k(nw_hbm, t_hbm, idx_hbm, bt_hbm, seg_hbm, z_hbm, out_hbm,
          idx0, idx1, bt0, bt1, seg0, seg1, trow0, trow1, rows0, rows1,
          acc_sh, sem1, sem2):
        cid = lax.axis_index("c")
        sid = lax.axis_index("s")
        wid = cid * NS + sid
        r0 = sid * NPW
        pltpu.sync_copy(z_hbm.at[pl.ds(r0, NPW)], acc_sh.at[pl.ds(r0, NPW)])
        plsc.subcore_barrier()

        bufs = [(idx0, bt0, seg0, trow0, rows0), (idx1, bt1, seg1, trow1, rows1)]

        def meta(g, b):
            base = wid * EPW + g * C
            pltpu.sync_copy(idx_hbm.at[pl.ds(base, C)], b[0])
            pltpu.sync_copy(bt_hbm.at[pl.ds(base, C)], b[1])
            pltpu.sync_copy(seg_hbm.at[pl.ds(base, C)], b[2])

        def fire(b):
            pltpu.async_copy(t_hbm.at[b[1]], b[3], sem1)
            pltpu.async_copy(nw_hbm.at[b[0]], b[4], sem2)

        def drain(b):
            pltpu.make_async_copy(t_hbm.at[b[1]], b[3], sem1).wait()
            pltpu.make_async_copy(nw_hbm.at[b[0]], b[4], sem2).wait()

        def compute(b):
            trow, rows = b[3], b[4]

            def edge(e, _):
                for kk in range(D // 16):
                    sl = pl.ds(kk * 16, 16)
                    x = rows[e, sl] + trow[e, sl]
                    rows[e, sl] = jnp.maximum(x, NEG_SLOPE * x)
                return 0

            lax.fori_loop(0, C, edge, 0, unroll=4)
            pltpu.sync_copy(rows, acc_sh.at[b[2]], add=True)

        meta(0, bufs[0])
        fire(bufs[0])

        def pairbody(gp, _):
            g0 = 2 * gp
            meta(g0 + 1, bufs[1])
            drain(bufs[0])
            fire(bufs[1])
            compute(bufs[0])

            @pl.when(g0 + 2 < n_chunks)
            def _():
                meta(g0 + 2, bufs[0])

            drain(bufs[1])

            @pl.when(g0 + 2 < n_chunks)
            def _():
                fire(bufs[0])

            compute(bufs[1])
            return 0

        lax.fori_loop(0, n_chunks // 2, pairbody, 0)
        if n_chunks % 2 == 1:
            drain(bufs[0])
            compute(bufs[0])
        plsc.subcore_barrier()
        pltpu.sync_copy(acc_sh.at[pl.ds(r0, NPW)],
                        out_hbm.at[cid, pl.ds(r0, NPW)])

    return k


def _make_score_kernel(Nq, Nk, E, C, scale):
    EPW = E // NW
    assert EPW % C == 0
    n_chunks = EPW // C

    @functools.partial(
        pl.kernel,
        mesh=_mesh(),
        compiler_params=pltpu.CompilerParams(needs_layout_passes=False),
        out_type=jax.ShapeDtypeStruct((E,), jnp.float32),
        scratch_types=[
            pltpu.VMEM((C,), jnp.int32), pltpu.VMEM((C,), jnp.int32),
            pltpu.VMEM((C,), jnp.int32), pltpu.VMEM((C,), jnp.int32),
            pltpu.VMEM((C, D), jnp.float32), pltpu.VMEM((C, D), jnp.float32),
            pltpu.VMEM((C, D), jnp.float32), pltpu.VMEM((C, D), jnp.float32),
            pltpu.VMEM((C, 16), jnp.float32),
            pltpu.VMEM((C,), jnp.float32),
            pltpu.SemaphoreType.DMA, pltpu.SemaphoreType.DMA,
        ],
    )
    def k(q_hbm, kk_hbm, seg_hbm, idx_hbm, sc_hbm,
          seg0, seg1, idx0, idx1, qrow0, qrow1, krow0, krow1, sum_v, sc_v,
          sem1, sem2):
        cid = lax.axis_index("c")
        sid = lax.axis_index("s")
        wid = cid * NS + sid

        bufs = [(seg0, idx0, qrow0, krow0), (seg1, idx1, qrow1, krow1)]
        lane = lax.iota(jnp.int32, 16)

        def meta(g, b):
            base = wid * EPW + g * C
            pltpu.sync_copy(seg_hbm.at[pl.ds(base, C)], b[0])
            pltpu.sync_copy(idx_hbm.at[pl.ds(base, C)], b[1])

        def fire(b):
            pltpu.async_copy(q_hbm.at[b[0]], b[2], sem1)
            pltpu.async_copy(kk_hbm.at[b[1]], b[3], sem2)

        def drain(b):
            pltpu.make_async_copy(q_hbm.at[b[0]], b[2], sem1).wait()
            pltpu.make_async_copy(kk_hbm.at[b[1]], b[3], sem2).wait()

        def compute(g, b):
            base = wid * EPW + g * C
            qrow, krow = b[2], b[3]

            def pair(p, _):
                acc = qrow[p, pl.ds(0, 16)] * krow[p, pl.ds(0, 16)]
                for kk in range(1, D // 16):
                    sl = pl.ds(kk * 16, 16)
                    acc = acc + qrow[p, sl] * krow[p, sl]
                sum_v[p, pl.ds(0, 16)] = acc
                return 0

            lax.fori_loop(0, C, pair, 0, unroll=4)

            def grp(g16, _):
                rows = g16 * 16 + lane
                tot = plsc.load_gather(sum_v, [rows, jnp.zeros((16,), jnp.int32)])
                for j in range(1, 16):
                    tot = tot + plsc.load_gather(
                        sum_v, [rows, jnp.full((16,), j, jnp.int32)])
                sc_v[pl.ds(g16 * 16, 16)] = tot * scale
                return 0

            lax.fori_loop(0, C // 16, grp, 0)
            pltpu.sync_copy(sc_v, sc_hbm.at[pl.ds(base, C)])

        meta(0, bufs[0])
        fire(bufs[0])

        def pairbody(gp, _):
            g0 = 2 * gp
            meta(g0 + 1, bufs[1])
            drain(bufs[0])
            fire(bufs[1])
            compute(g0, bufs[0])

            @pl.when(g0 + 2 < n_chunks)
            def _():
                meta(g0 + 2, bufs[0])

            drain(bufs[1])

            @pl.when(g0 + 2 < n_chunks)
            def _():
                fire(bufs[0])

            compute(g0 + 1, bufs[1])
            return 0

        lax.fori_loop(0, n_chunks // 2, pairbody, 0)
        if n_chunks % 2 == 1:
            drain(bufs[0])
            compute(n_chunks - 1, bufs[0])

    return k


def _make_attn_acc_kernel(Nq, Nv, E, C):
    EPW = E // NW
    NPW = Nq // NS
    assert NPW % 8 == 0 and EPW % C == 0
    n_chunks = EPW // C

    @functools.partial(
        pl.kernel,
        mesh=_mesh(),
        compiler_params=pltpu.CompilerParams(needs_layout_passes=False),
        out_type=[
            jax.ShapeDtypeStruct((NC, Nq, D), jnp.float32),
            jax.ShapeDtypeStruct((E,), jnp.float32),
        ],
        scratch_types=[
            pltpu.VMEM((C,), jnp.int32), pltpu.VMEM((C,), jnp.int32),
            pltpu.VMEM((C,), jnp.int32), pltpu.VMEM((C,), jnp.int32),
            pltpu.VMEM((C,), jnp.int32), pltpu.VMEM((C,), jnp.int32),
            pltpu.VMEM((C, D), jnp.float32), pltpu.VMEM((C, D), jnp.float32),
            pltpu.VMEM((C, D), jnp.float32), pltpu.VMEM((C, D), jnp.float32),
            pltpu.VMEM((C,), jnp.float32), pltpu.VMEM((C,), jnp.float32),
            pltpu.VMEM_SHARED((Nq, D), jnp.float32),
            pltpu.SemaphoreType.DMA, pltpu.SemaphoreType.DMA,
        ],
    )
    def k(v_hbm, sc_hbm, m2d_hbm, seg_hbm, idx_hbm, z_hbm,
          numer_hbm, ex_hbm,
          seg0, seg1, idx0, idx1, rix0, rix1, vrow0, vrow1, mrow0, mrow1,
          sc0, sc1, num_sh, sem1, sem2):
        cid = lax.axis_index("c")
        sid = lax.axis_index("s")
        wid = cid * NS + sid
        r0 = sid * NPW
        pltpu.sync_copy(z_hbm.at[pl.ds(r0, NPW)], num_sh.at[pl.ds(r0, NPW)])
        plsc.subcore_barrier()

        bufs = [(seg0, idx0, rix0, vrow0, mrow0, sc0),
                (seg1, idx1, rix1, vrow1, mrow1, sc1)]
        lane = lax.iota(jnp.int32, 16)

        def meta(g, b):
            base = wid * EPW + g * C
            pltpu.sync_copy(seg_hbm.at[pl.ds(base, C)], b[0])
            pltpu.sync_copy(idx_hbm.at[pl.ds(base, C)], b[1])
            pltpu.sync_copy(sc_hbm.at[pl.ds(base, C)], b[5])

            def rx(g16, _):
                sl = pl.ds(g16 * 16, 16)
                b[2][sl] = lax.shift_right_logical(b[0][sl], 7)
                return 0

            lax.fori_loop(0, C // 16, rx, 0)

        def fire(b):
            pltpu.async_copy(v_hbm.at[b[1]], b[3], sem1)
            pltpu.async_copy(m2d_hbm.at[b[2]], b[4], sem2)

        def drain(b):
            pltpu.make_async_copy(v_hbm.at[b[1]], b[3], sem1).wait()
            pltpu.make_async_copy(m2d_hbm.at[b[2]], b[4], sem2).wait()

        def compute(g, b):
            base = wid * EPW + g * C
            seg_v, vrow, mrow, sc_v = b[0], b[3], b[4], b[5]

            def grp(g16, _):
                rows = g16 * 16 + lane
                segv = seg_v[pl.ds(g16 * 16, 16)]
                col = jnp.bitwise_and(segv, 127)
                mvec = plsc.load_gather(mrow, [rows, col])
                svec = sc_v[pl.ds(g16 * 16, 16)]
                sc_v[pl.ds(g16 * 16, 16)] = jnp.exp(svec - mvec)
                return 0

            lax.fori_loop(0, C // 16, grp, 0)

            def pair(p, _):
                ex = plsc.load_gather(sc_v, [jnp.full((16,), p, jnp.int32)])
                for kk in range(D // 16):
                    sl = pl.ds(kk * 16, 16)
                    vrow[p, sl] = vrow[p, sl] * ex
                return 0

            lax.fori_loop(0, C, pair, 0, unroll=4)
            pltpu.sync_copy(vrow, num_sh.at[seg_v], add=True)
            pltpu.sync_copy(sc_v, ex_hbm.at[pl.ds(base, C)])

        meta(0, bufs[0])
        fire(bufs[0])

        def pairbody(gp, _):
            g0 = 2 * gp
            meta(g0 + 1, bufs[1])
            drain(bufs[0])
            fire(bufs[1])
            compute(g0, bufs[0])

            @pl.when(g0 + 2 < n_chunks)
            def _():
                meta(g0 + 2, bufs[0])

            drain(bufs[1])

            @pl.when(g0 + 2 < n_chunks)
            def _():
                fire(bufs[0])

            compute(g0 + 1, bufs[1])
            return 0

        lax.fori_loop(0, n_chunks // 2, pairbody, 0)
        if n_chunks % 2 == 1:
            drain(bufs[0])
            compute(n_chunks - 1, bufs[0])
        plsc.subcore_barrier()
        pltpu.sync_copy(num_sh.at[pl.ds(r0, NPW)],
                        numer_hbm.at[cid, pl.ds(r0, NPW)])

    return k


def _pair_select_body(wp1_ref, bp1_ref, wp2_ref, bp2_ref, out_ref):
    rows = out_ref.shape[0]
    ind = (lax.broadcasted_iota(jnp.int32, (rows, BS), 0) * BS +
           lax.broadcasted_iota(jnp.int32, (rows, BS), 1)).astype(jnp.float32)
    h = jnp.maximum(ind @ wp1_ref[...] + bp1_ref[...], 0.0)
    o = h @ wp2_ref[...] + bp2_ref[...]
    o3 = o.reshape(rows, BS, BS)
    out_ref[...] = jnp.argmax(o3, axis=2).astype(jnp.int32)


def _pair_select(Wp1, bp1, Wp2, bp2, n1):
    rows = -(-n1 // BS)
    preds2d = pl.pallas_call(
        _pair_select_body,
        out_shape=jax.ShapeDtypeStruct((rows, BS), jnp.int32),
    )(Wp1, bp1[None, :], Wp2, bp2[None, :])
    return preds2d.reshape(rows * BS)


def _leaky(x):
    return jnp.maximum(x, NEG_SLOPE * x)


def _onehot(idx, n):
    return (idx[:, None] == jnp.arange(n, dtype=idx.dtype)[None, :]).astype(jnp.float32)


def kernel(seg_m1, atom_type1, atom_feat1, bond_type1, inn_seg_i1, inn_idx_j1,
           out_seg_i1, out_idx_j1, seg_m2, atom_type2, atom_feat2, bond_type2,
           inn_seg_i2, inn_idx_j2, out_seg_i2, out_idx_j2,
           atom_emb, bond_emb, W_atom, b_atom, Wn, We, bm, Wq, Wk, Wv,
           W_ro, b_ro, W_lbl, b_lbl, Wp1, bp1, Wp2, bp2):
    n1 = seg_m1.shape[0]
    E1 = inn_seg_i1.shape[0]
    P = out_seg_i1.shape[0]
    preds = _pair_select(Wp1, bp1, Wp2, bp2, n1)
    Ln2 = preds.shape[0]
    Ln1 = n1
    scale = 1.0 / np.sqrt(float(D))
    N_STEP = Wn.shape[0]

    oh_preds = _onehot(preds, BS)
    counts = oh_preds.sum(axis=0)
    seg2i = inn_seg_i2[:BS]
    idx2j = inn_idx_j2[:BS]
    bt2 = bond_type2[:BS]
    os2 = out_seg_i2[:BS]
    oj2 = out_idx_j2[:BS]

    node1 = jnp.concatenate([_onehot(atom_type1, atom_emb.shape[0]) @ atom_emb,
                             atom_feat1], axis=-1) @ W_atom + b_atom
    node2_tab = jnp.concatenate([_onehot(atom_type2[:BS], atom_emb.shape[0]) @ atom_emb,
                                 atom_feat2[:BS]], axis=-1) @ W_atom + b_atom
    node2 = oh_preds @ node2_tab

    G_idx2j = _onehot(idx2j, Ln2)
    G_os2 = _onehot(os2, Ln2)
    G_oj2 = _onehot(oj2, Ln1)
    OH_bt2 = _onehot(bt2, bond_emb.shape[0])
    S_in = _onehot(seg2i, Ln2)
    M2 = (os2[:, None] == os2[None, :])

    NPAD = ((Ln1 + NS * 8 - 1) // (NS * 8)) * NS * 8
    zeros128 = jnp.zeros((NPAD, D), jnp.float32)

    inner_k = _make_inner_kernel(NPAD, E1, 80)
    score_k = _make_score_kernel(Ln1, Ln2, P, 200, scale)
    attn_k = _make_attn_acc_kernel(NPAD, Ln2, P, 80)

    for s in range(N_STEP):
        nW1 = node1 @ Wn[s]
        t1 = bond_emb @ We[s] + bm[s]
        inner_pl = inner_k(nW1, t1, inn_idx_j1, bond_type1, inn_seg_i1, zeros128)
        inner1 = (inner_pl[0] + inner_pl[1])[:Ln1]

        q1 = node1 @ Wq[s]; k1 = node1 @ Wk[s]; v1 = node1 @ Wv[s]
        k2 = node2 @ Wk[s]; v2 = node2 @ Wv[s]
        sc1 = score_k(q1, k2, out_seg_i1, out_idx_j1)
        m1 = jax.ops.segment_max(sc1, out_seg_i1, num_segments=Ln1,
                                 indices_are_sorted=True)
        m1 = jnp.where(jnp.isfinite(m1), m1, 0.0)
        m2d = jnp.pad(m1, (0, NPAD - Ln1)).reshape(NPAD // 128, 128)
        numer_pl, ex1 = attn_k(v2, sc1, m2d, out_seg_i1, out_idx_j1, zeros128)
        denom = jax.ops.segment_sum(ex1, out_seg_i1, num_segments=Ln1,
                                    indices_are_sorted=True)
        outer1 = ((numer_pl[0, :Ln1] + numer_pl[1, :Ln1])
                  / (denom[:, None] + 1e-16))

        t2 = bond_emb @ We[s] + bm[s]
        nW2rows = (G_idx2j @ node2) @ Wn[s]
        msg2rows = _leaky(nW2rows + OH_bt2 @ t2)
        inner2 = S_in.T @ (counts[:, None] * msg2rows)

        q2rows = (G_os2 @ node2) @ Wq[s]
        k1rows = G_oj2 @ k1
        v1rows = G_oj2 @ v1
        sc2 = jnp.sum(q2rows * k1rows, axis=-1) * scale
        m2 = jnp.max(jnp.where(M2, sc2[None, :], -jnp.inf), axis=1)
        ex2 = jnp.exp(sc2 - m2)
        s2 = M2.astype(jnp.float32) @ (counts * ex2)
        w2 = counts * ex2 / (s2 + 1e-16)
        outer2 = G_os2.T @ (w2[:, None] * v1rows)

        node1 = node1 + inner1 + outer1
        node2 = node2 + inner2 + outer2

    OH_m1 = _onehot(seg_m1, N_MOL)
    OH_m2 = oh_preds @ _onehot(seg_m2[:BS], N_MOL)
    d1_vec = OH_m1.T @ _leaky(node1 @ W_ro + b_ro)
    d2_vec = OH_m2.T @ _leaky(node2 @ W_ro + b_ro)
    pred1 = d1_vec @ W_lbl + b_lbl
    pred2 = d2_vec @ W_lbl + b_lbl
    return (pred1, pred2, preds)


# speedup vs baseline: 1.0251x; 1.0251x over previous
import functools

import jax
import jax.numpy as jnp
import numpy as np
from jax import lax
from jax.experimental import pallas as pl
from jax.experimental.pallas import tpu as pltpu
from jax.experimental.pallas import tpu_sc as plsc

N_MOL = 256
BS = 128
D = 128
NEG_SLOPE = 0.01
NC = 2
NS = 16
NW = NC * NS


def _mesh():
    return plsc.VectorSubcoreMesh(core_axis_name="c", subcore_axis_name="s")


def _make_inner_kernel(N, E, C):
    EPW = E // NW
    NPW = N // NS
    assert NPW % 8 == 0 and EPW % C == 0
    n_chunks = EPW // C

    @functools.partial(
        pl.kernel,
        mesh=_mesh(),
        compiler_params=pltpu.CompilerParams(needs_layout_passes=False),
        out_type=jax.ShapeDtypeStruct((NC, N, D), jnp.float32),
        scratch_types=[
            pltpu.VMEM((C,), jnp.int32), pltpu.VMEM((C,), jnp.int32),
            pltpu.VMEM((C,), jnp.int32), pltpu.VMEM((C,), jnp.int32),
            pltpu.VMEM((C,), jnp.int32), pltpu.VMEM((C,), jnp.int32),
            pltpu.VMEM((C, D), jnp.float32), pltpu.VMEM((C, D), jnp.float32),
            pltpu.VMEM((C, D), jnp.float32), pltpu.VMEM((C, D), jnp.float32),
            pltpu.VMEM_SHARED((N, D), jnp.float32),
            pltpu.SemaphoreType.DMA, pltpu.SemaphoreType.DMA,
        ],
    )
    def k(nw_hbm, t_hbm, idx_hbm, bt_hbm, seg_hbm, z_hbm, out_hbm,
          idx0, idx1, bt0, bt1, seg0, seg1, trow0, trow1, rows0, rows1,
          acc_sh, sem1, sem2):
        cid = lax.axis_index("c")
        sid = lax.axis_index("s")
        wid = cid * NS + sid
        r0 = sid * NPW
        pltpu.sync_copy(z_hbm.at[pl.ds(r0, NPW)], acc_sh.at[pl.ds(r0, NPW)])
        plsc.subcore_barrier()

        bufs = [(idx0, bt0, seg0, trow0, rows0), (idx1, bt1, seg1, trow1, rows1)]

        def meta(g, b):
            base = wid * EPW + g * C
            pltpu.sync_copy(idx_hbm.at[pl.ds(base, C)], b[0])
            pltpu.sync_copy(bt_hbm.at[pl.ds(base, C)], b[1])
            pltpu.sync_copy(seg_hbm.at[pl.ds(base, C)], b[2])

        def fire(b):
            pltpu.async_copy(t_hbm.at[b[1]], b[3], sem1)
            pltpu.async_copy(nw_hbm.at[b[0]], b[4], sem2)

        def drain(b):
            pltpu.make_async_copy(t_hbm.at[b[1]], b[3], sem1).wait()
            pltpu.make_async_copy(nw_hbm.at[b[0]], b[4], sem2).wait()

        def compute(b):
            trow, rows = b[3], b[4]

            def edge(e, _):
                for kk in range(D // 16):
                    sl = pl.ds(kk * 16, 16)
                    x = rows[e, sl] + trow[e, sl]
                    rows[e, sl] = jnp.maximum(x, NEG_SLOPE * x)
                return 0

            lax.fori_loop(0, C, edge, 0, unroll=2)
            pltpu.sync_copy(rows, acc_sh.at[b[2]], add=True)

        meta(0, bufs[0])
        fire(bufs[0])

        def pairbody(gp, _):
            g0 = 2 * gp
            meta(g0 + 1, bufs[1])
            drain(bufs[0])
            fire(bufs[1])
            compute(bufs[0])

            @pl.when(g0 + 2 < n_chunks)
            def _():
                meta(g0 + 2, bufs[0])

            drain(bufs[1])

            @pl.when(g0 + 2 < n_chunks)
            def _():
                fire(bufs[0])

            compute(bufs[1])
            return 0

        lax.fori_loop(0, n_chunks // 2, pairbody, 0)
        if n_chunks % 2 == 1:
            drain(bufs[0])
            compute(bufs[0])
        plsc.subcore_barrier()
        pltpu.sync_copy(acc_sh.at[pl.ds(r0, NPW)],
                        out_hbm.at[cid, pl.ds(r0, NPW)])

    return k


def _make_score_kernel(Nq, Nk, E, C, scale):
    EPW = E // NW
    assert EPW % C == 0
    n_chunks = EPW // C

    @functools.partial(
        pl.kernel,
        mesh=_mesh(),
        compiler_params=pltpu.CompilerParams(needs_layout_passes=False),
        out_type=jax.ShapeDtypeStruct((E,), jnp.float32),
        scratch_types=[
            pltpu.VMEM((C,), jnp.int32), pltpu.VMEM((C,), jnp.int32),
            pltpu.VMEM((C,), jnp.int32), pltpu.VMEM((C,), jnp.int32),
            pltpu.VMEM((C, D), jnp.float32), pltpu.VMEM((C, D), jnp.float32),
            pltpu.VMEM((C, D), jnp.float32), pltpu.VMEM((C, D), jnp.float32),
            pltpu.VMEM((C, 16), jnp.float32),
            pltpu.VMEM((C,), jnp.float32),
            pltpu.SemaphoreType.DMA, pltpu.SemaphoreType.DMA,
        ],
    )
    def k(q_hbm, kk_hbm, seg_hbm, idx_hbm, sc_hbm,
          seg0, seg1, idx0, idx1, qrow0, qrow1, krow0, krow1, sum_v, sc_v,
          sem1, sem2):
        cid = lax.axis_index("c")
        sid = lax.axis_index("s")
        wid = cid * NS + sid

        bufs = [(seg0, idx0, qrow0, krow0), (seg1, idx1, qrow1, krow1)]
        lane = lax.iota(jnp.int32, 16)

        def meta(g, b):
            base = wid * EPW + g * C
            pltpu.sync_copy(seg_hbm.at[pl.ds(base, C)], b[0])
            pltpu.sync_copy(idx_hbm.at[pl.ds(base, C)], b[1])

        def fire(b):
            pltpu.async_copy(q_hbm.at[b[0]], b[2], sem1)
            pltpu.async_copy(kk_hbm.at[b[1]], b[3], sem2)

        def drain(b):
            pltpu.make_async_copy(q_hbm.at[b[0]], b[2], sem1).wait()
            pltpu.make_async_copy(kk_hbm.at[b[1]], b[3], sem2).wait()

        def compute(g, b):
            base = wid * EPW + g * C
            qrow, krow = b[2], b[3]

            def pair(p, _):
                acc = qrow[p, pl.ds(0, 16)] * krow[p, pl.ds(0, 16)]
                for kk in range(1, D // 16):
                    sl = pl.ds(kk * 16, 16)
                    acc = acc + qrow[p, sl] * krow[p, sl]
                sum_v[p, pl.ds(0, 16)] = acc
                return 0

            lax.fori_loop(0, C, pair, 0, unroll=2)

            def grp(g16, _):
                rows = g16 * 16 + lane
                tot = plsc.load_gather(sum_v, [rows, jnp.zeros((16,), jnp.int32)])
                for j in range(1, 16):
                    tot = tot + plsc.load_gather(
                        sum_v, [rows, jnp.full((16,), j, jnp.int32)])
                sc_v[pl.ds(g16 * 16, 16)] = tot * scale
                return 0

            lax.fori_loop(0, C // 16, grp, 0)
            pltpu.sync_copy(sc_v, sc_hbm.at[pl.ds(base, C)])

        meta(0, bufs[0])
        fire(bufs[0])

        def pairbody(gp, _):
            g0 = 2 * gp
            meta(g0 + 1, bufs[1])
            drain(bufs[0])
            fire(bufs[1])
            compute(g0, bufs[0])

            @pl.when(g0 + 2 < n_chunks)
            def _():
                meta(g0 + 2, bufs[0])

            drain(bufs[1])

            @pl.when(g0 + 2 < n_chunks)
            def _():
                fire(bufs[0])

            compute(g0 + 1, bufs[1])
            return 0

        lax.fori_loop(0, n_chunks // 2, pairbody, 0)
        if n_chunks % 2 == 1:
            drain(bufs[0])
            compute(n_chunks - 1, bufs[0])

    return k


def _make_attn_acc_kernel(Nq, Nv, E, C):
    EPW = E // NW
    NPW = Nq // NS
    assert NPW % 8 == 0 and EPW % C == 0
    n_chunks = EPW // C

    @functools.partial(
        pl.kernel,
        mesh=_mesh(),
        compiler_params=pltpu.CompilerParams(needs_layout_passes=False),
        out_type=[
            jax.ShapeDtypeStruct((NC, Nq, D), jnp.float32),
            jax.ShapeDtypeStruct((E,), jnp.float32),
        ],
        scratch_types=[
            pltpu.VMEM((C,), jnp.int32), pltpu.VMEM((C,), jnp.int32),
            pltpu.VMEM((C,), jnp.int32), pltpu.VMEM((C,), jnp.int32),
            pltpu.VMEM((C,), jnp.int32), pltpu.VMEM((C,), jnp.int32),
            pltpu.VMEM((C, D), jnp.float32), pltpu.VMEM((C, D), jnp.float32),
            pltpu.VMEM((C, D), jnp.float32), pltpu.VMEM((C, D), jnp.float32),
            pltpu.VMEM((C,), jnp.float32), pltpu.VMEM((C,), jnp.float32),
            pltpu.VMEM_SHARED((Nq, D), jnp.float32),
            pltpu.SemaphoreType.DMA, pltpu.SemaphoreType.DMA,
        ],
    )
    def k(v_hbm, sc_hbm, m2d_hbm, seg_hbm, idx_hbm, z_hbm,
          numer_hbm, ex_hbm,
          seg0, seg1, idx0, idx1, rix0, rix1, vrow0, vrow1, mrow0, mrow1,
          sc0, sc1, num_sh, sem1, sem2):
        cid = lax.axis_index("c")
        sid = lax.axis_index("s")
        wid = cid * NS + sid
        r0 = sid * NPW
        pltpu.sync_copy(z_hbm.at[pl.ds(r0, NPW)], num_sh.at[pl.ds(r0, NPW)])
        plsc.subcore_barrier()

        bufs = [(seg0, idx0, rix0, vrow0, mrow0, sc0),
                (seg1, idx1, rix1, vrow1, mrow1, sc1)]
        lane = lax.iota(jnp.int32, 16)

        def meta(g, b):
            base = wid * EPW + g * C
            pltpu.sync_copy(seg_hbm.at[pl.ds(base, C)], b[0])
            pltpu.sync_copy(idx_hbm.at[pl.ds(base, C)], b[1])
            pltpu.sync_copy(sc_hbm.at[pl.ds(base, C)], b[5])

            def rx(g16, _):
                sl = pl.ds(g16 * 16, 16)
                b[2][sl] = lax.shift_right_logical(b[0][sl], 7)
                return 0

            lax.fori_loop(0, C // 16, rx, 0)

        def fire(b):
            pltpu.async_copy(v_hbm.at[b[1]], b[3], sem1)
            pltpu.async_copy(m2d_hbm.at[b[2]], b[4], sem2)

        def drain(b):
            pltpu.make_async_copy(v_hbm.at[b[1]], b[3], sem1).wait()
            pltpu.make_async_copy(m2d_hbm.at[b[2]], b[4], sem2).wait()

        def compute(g, b):
            base = wid * EPW + g * C
            seg_v, vrow, mrow, sc_v = b[0], b[3], b[4], b[5]

            def grp(g16, _):
                rows = g16 * 16 + lane
                segv = seg_v[pl.ds(g16 * 16, 16)]
                col = jnp.bitwise_and(segv, 127)
                mvec = plsc.load_gather(mrow, [rows, col])
                svec = sc_v[pl.ds(g16 * 16, 16)]
                sc_v[pl.ds(g16 * 16, 16)] = jnp.exp(svec - mvec)
                return 0

            lax.fori_loop(0, C // 16, grp, 0)

            def pair(p, _):
                ex = plsc.load_gather(sc_v, [jnp.full((16,), p, jnp.int32)])
                for kk in range(D // 16):
                    sl = pl.ds(kk * 16, 16)
                    vrow[p, sl] = vrow[p, sl] * ex
                return 0

            lax.fori_loop(0, C, pair, 0, unroll=2)
            pltpu.sync_copy(vrow, num_sh.at[seg_v], add=True)
            pltpu.sync_copy(sc_v, ex_hbm.at[pl.ds(base, C)])

        meta(0, bufs[0])
        fire(bufs[0])

        def pairbody(gp, _):
            g0 = 2 * gp
            meta(g0 + 1, bufs[1])
            drain(bufs[0])
            fire(bufs[1])
            compute(g0, bufs[0])

            @pl.when(g0 + 2 < n_chunks)
            def _():
                meta(g0 + 2, bufs[0])

            drain(bufs[1])

            @pl.when(g0 + 2 < n_chunks)
            def _():
                fire(bufs[0])

            compute(g0 + 1, bufs[1])
            return 0

        lax.fori_loop(0, n_chunks // 2, pairbody, 0)
        if n_chunks % 2 == 1:
            drain(bufs[0])
            compute(n_chunks - 1, bufs[0])
        plsc.subcore_barrier()
        pltpu.sync_copy(num_sh.at[pl.ds(r0, NPW)],
                        numer_hbm.at[cid, pl.ds(r0, NPW)])

    return k


def _pair_select_body(wp1_ref, bp1_ref, wp2_ref, bp2_ref, out_ref):
    rows = out_ref.shape[0]
    ind = (lax.broadcasted_iota(jnp.int32, (rows, BS), 0) * BS +
           lax.broadcasted_iota(jnp.int32, (rows, BS), 1)).astype(jnp.float32)
    h = jnp.maximum(ind @ wp1_ref[...] + bp1_ref[...], 0.0)
    o = h @ wp2_ref[...] + bp2_ref[...]
    o3 = o.reshape(rows, BS, BS)
    out_ref[...] = jnp.argmax(o3, axis=2).astype(jnp.int32)


def _pair_select(Wp1, bp1, Wp2, bp2, n1):
    rows = -(-n1 // BS)
    preds2d = pl.pallas_call(
        _pair_select_body,
        out_shape=jax.ShapeDtypeStruct((rows, BS), jnp.int32),
    )(Wp1, bp1[None, :], Wp2, bp2[None, :])
    return preds2d.reshape(rows * BS)


def _leaky(x):
    return jnp.maximum(x, NEG_SLOPE * x)


def _onehot(idx, n):
    return (idx[:, None] == jnp.arange(n, dtype=idx.dtype)[None, :]).astype(jnp.float32)


def kernel(seg_m1, atom_type1, atom_feat1, bond_type1, inn_seg_i1, inn_idx_j1,
           out_seg_i1, out_idx_j1, seg_m2, atom_type2, atom_feat2, bond_type2,
           inn_seg_i2, inn_idx_j2, out_seg_i2, out_idx_j2,
           atom_emb, bond_emb, W_atom, b_atom, Wn, We, bm, Wq, Wk, Wv,
           W_ro, b_ro, W_lbl, b_lbl, Wp1, bp1, Wp2, bp2):
    n1 = seg_m1.shape[0]
    E1 = inn_seg_i1.shape[0]
    P = out_seg_i1.shape[0]
    preds = _pair_select(Wp1, bp1, Wp2, bp2, n1)
    Ln2 = preds.shape[0]
    Ln1 = n1
    scale = 1.0 / np.sqrt(float(D))
    N_STEP = Wn.shape[0]

    oh_preds = _onehot(preds, BS)
    counts = oh_preds.sum(axis=0)
    seg2i = inn_seg_i2[:BS]
    idx2j = inn_idx_j2[:BS]
    bt2 = bond_type2[:BS]
    os2 = out_seg_i2[:BS]
    oj2 = out_idx_j2[:BS]

    node1 = jnp.concatenate([_onehot(atom_type1, atom_emb.shape[0]) @ atom_emb,
                             atom_feat1], axis=-1) @ W_atom + b_atom
    node2_tab = jnp.concatenate([_onehot(atom_type2[:BS], atom_emb.shape[0]) @ atom_emb,
                                 atom_feat2[:BS]], axis=-1) @ W_atom + b_atom
    node2 = oh_preds @ node2_tab

    G_idx2j = _onehot(idx2j, Ln2)
    G_os2 = _onehot(os2, Ln2)
    G_oj2 = _onehot(oj2, Ln1)
    OH_bt2 = _onehot(bt2, bond_emb.shape[0])
    S_in = _onehot(seg2i, Ln2)
    M2 = (os2[:, None] == os2[None, :])

    NPAD = ((Ln1 + NS * 8 - 1) // (NS * 8)) * NS * 8
    zeros128 = jnp.zeros((NPAD, D), jnp.float32)

    inner_k = _make_inner_kernel(NPAD, E1, 80)
    score_k = _make_score_kernel(Ln1, Ln2, P, 200, scale)
    attn_k = _make_attn_acc_kernel(NPAD, Ln2, P, 80)

    for s in range(N_STEP):
        nW1 = node1 @ Wn[s]
        t1 = bond_emb @ We[s] + bm[s]
        inner_pl = inner_k(nW1, t1, inn_idx_j1, bond_type1, inn_seg_i1, zeros128)
        inner1 = (inner_pl[0] + inner_pl[1])[:Ln1]

        q1 = node1 @ Wq[s]; k1 = node1 @ Wk[s]; v1 = node1 @ Wv[s]
        k2 = node2 @ Wk[s]; v2 = node2 @ Wv[s]
        sc1 = score_k(q1, k2, out_seg_i1, out_idx_j1)
        m1 = jax.ops.segment_max(sc1, out_seg_i1, num_segments=Ln1,
                                 indices_are_sorted=True)
        m1 = jnp.where(jnp.isfinite(m1), m1, 0.0)
        m2d = jnp.pad(m1, (0, NPAD - Ln1)).reshape(NPAD // 128, 128)
        numer_pl, ex1 = attn_k(v2, sc1, m2d, out_seg_i1, out_idx_j1, zeros128)
        denom = jax.ops.segment_sum(ex1, out_seg_i1, num_segments=Ln1,
                                    indices_are_sorted=True)
        outer1 = ((numer_pl[0, :Ln1] + numer_pl[1, :Ln1])
                  / (denom[:, None] + 1e-16))

        t2 = bond_emb @ We[s] + bm[s]
        nW2rows = (G_idx2j @ node2) @ Wn[s]
        msg2rows = _leaky(nW2rows + OH_bt2 @ t2)
        inner2 = S_in.T @ (counts[:, None] * msg2rows)

        q2rows = (G_os2 @ node2) @ Wq[s]
        k1rows = G_oj2 @ k1
        v1rows = G_oj2 @ v1
        sc2 = jnp.sum(q2rows * k1rows, axis=-1) * scale
        m2 = jnp.max(jnp.where(M2, sc2[None, :], -jnp.inf), axis=1)
        ex2 = jnp.exp(sc2 - m2)
        s2 = M2.astype(jnp.float32) @ (counts * ex2)
        w2 = counts * ex2 / (s2 + 1e-16)
        outer2 = G_os2.T @ (w2[:, None] * v1rows)

        node1 = node1 + inner1 + outer1
        node2 = node2 + inner2 + outer2

    OH_m1 = _onehot(seg_m1, N_MOL)
    OH_m2 = oh_preds @ _onehot(seg_m2[:BS], N_MOL)
    d1_vec = OH_m1.T @ _leaky(node1 @ W_ro + b_ro)
    d2_vec = OH_m2.T @ _leaky(node2 @ W_ro + b_ro)
    pred1 = d1_vec @ W_lbl + b_lbl
    pred2 = d2_vec @ W_lbl + b_lbl
    return (pred1, pred2, preds)
